# baseline (speedup 1.0000x reference)
import functools
import math

import jax
import jax.numpy as jnp
from jax import lax
from jax.experimental import pallas as pl
from jax.experimental.pallas import tpu as pltpu
from jax.experimental.pallas import tpu_sc as plsc

DEPTH = 64
SCALE = math.sqrt(DEPTH)

NC = 2
NS = 16
NW = NC * NS
LANES = 16
CHUNK = 128


def _make_lookup(n_rows: int):
  assert n_rows % (NW * CHUNK) == 0
  rows_per_w = n_rows // NW
  n_chunks = rows_per_w // CHUNK
  mesh = plsc.VectorSubcoreMesh(core_axis_name="c", subcore_axis_name="s")

  @functools.partial(
      pl.kernel,
      mesh=mesh,
      out_type=jax.ShapeDtypeStruct((n_rows, DEPTH), jnp.float32),
      scratch_types=[
          pltpu.VMEM((n_chunks, CHUNK), jnp.int32),
          pltpu.VMEM((CHUNK, DEPTH), jnp.float32),
          pltpu.SemaphoreType.DMA,
      ],
      compiler_params=pltpu.CompilerParams(use_tc_tiling_on_sc=False),
  )
  def lookup(lut_hbm, idx_hbm, out_hbm, idx_v, buf, sem):
    wid = lax.axis_index("s") * NC + lax.axis_index("c")
    base = wid * rows_per_w
    pltpu.sync_copy(idx_hbm.at[wid], idx_v)

    def do_chunk(j, carry):
      pltpu.async_copy(lut_hbm.at[idx_v.at[j]], buf, sem).wait()

      def scale_row(r, c):
        for cc in range(DEPTH // LANES):
          sl = pl.ds(cc * LANES, LANES)
          buf[r, sl] = buf[r, sl] * SCALE
        return c

      lax.fori_loop(0, CHUNK, scale_row, 0, unroll=2)
      pltpu.sync_copy(buf, out_hbm.at[pl.ds(base + j * CHUNK, CHUNK)])
      return carry

    lax.fori_loop(0, n_chunks, do_chunk, 0)

  return lookup


def kernel(x, lut):
  b, t = x.shape
  n_rows = b * t
  idx = x.reshape(NW, n_rows // (NW * CHUNK), CHUNK).astype(jnp.int32)
  out = _make_lookup(n_rows)(lut, idx)
  return out.reshape(b, t, DEPTH)
